# Initial kernel scaffold; baseline (speedup 1.0000x reference)
import jax
import jax.numpy as jnp
from jax.experimental import pallas as pl


def kernel(target_item, history_sequence, mask, W_q, W_k, W_v, W_o):
    raise NotImplementedError("write your pallas kernel here")


# speedup vs baseline: 1.0075x; 1.0075x over previous
import jax
import jax.numpy as jnp
from jax.experimental import pallas as pl
from jax.experimental.pallas import tpu as pltpu

INPUT_DIM = 64
ATTN_DIM = 128
SCALE = ATTN_DIM ** 0.5
LC = 8


def _attn_kernel(tgt_ref, hist_ref, mask_ref, wq_ref, wk_ref, wv_ref, wo_ref,
                 out_ref):
    tgt = tgt_ref[...]
    Bb = tgt.shape[0]
    L = hist_ref.shape[1]
    A = jnp.dot(wq_ref[...], wk_ref[...].T, preferred_element_type=jnp.float32)
    Wvo = jnp.dot(wv_ref[...], wo_ref[...], preferred_element_type=jnp.float32)
    qp = (jnp.dot(tgt, A, preferred_element_type=jnp.float32)
          * (1.0 / SCALE))[:, None, :]

    def body(i, ctx):
        lc = pl.multiple_of(i * LC, LC)
        h = hist_ref[:, pl.ds(lc, LC), :]
        m = mask_ref[:, i, :, :]
        s = jnp.sum(h * qp, axis=2)
        s = s * m.reshape(Bb, LC).astype(jnp.float32)
        a = jnp.maximum(s, 0.0)
        return ctx + jnp.sum(a[:, :, None] * h, axis=1)

    ctx = jax.lax.fori_loop(0, L // LC, body, jnp.zeros_like(tgt))
    out_ref[...] = tgt + jnp.dot(ctx, Wvo, preferred_element_type=jnp.float32)


def kernel(target_item, history_sequence, mask, W_q, W_k, W_v, W_o):
    B, L, D = history_sequence.shape
    Bb = 128
    nc = L // LC
    mask4 = mask.reshape(B, nc, 1, LC)
    grid = (B // Bb,)
    return pl.pallas_call(
        _attn_kernel,
        grid=grid,
        in_specs=[
            pl.BlockSpec((Bb, D), lambda i: (i, 0)),
            pl.BlockSpec((Bb, L, D), lambda i: (i, 0, 0)),
            pl.BlockSpec((Bb, nc, 1, LC), lambda i: (i, 0, 0, 0)),
            pl.BlockSpec((D, ATTN_DIM), lambda i: (0, 0)),
            pl.BlockSpec((D, ATTN_DIM), lambda i: (0, 0)),
            pl.BlockSpec((D, ATTN_DIM), lambda i: (0, 0)),
            pl.BlockSpec((ATTN_DIM, D), lambda i: (0, 0)),
        ],
        out_specs=pl.BlockSpec((Bb, D), lambda i: (i, 0)),
        out_shape=jax.ShapeDtypeStruct((B, D), jnp.float32),
        compiler_params=pltpu.CompilerParams(
            dimension_semantics=("parallel",),
        ),
    )(target_item, history_sequence, mask4, W_q, W_k, W_v, W_o)
